# baseline (speedup 1.0000x reference)
import jax
import jax.numpy as jnp
from jax import lax
from jax.experimental import pallas as pl
from jax.experimental.pallas import tpu as pltpu
from jax.experimental.pallas import tpu_sc as plsc
from jax._src.pallas import mpmd as _mpmd

NX = 512
NY = 512
C = 64
P = 100000
B = 2
SPATIAL = B * NY * NX
NBIN = 2
CANVAS_ROWS = NBIN * SPATIAL

NC = 2
NS = 16
NW = NC * NS
PER_W = 3136
CH = 128
NCHUNK = 25


def _sc_scatter_body(f0, v0, f1, v1, zeros_in, canvas,
                     coords_v, idx_v, feat_v, sem_l, sem_s):
  del zeros_in
  wid = lax.axis_index("s") * NC + lax.axis_index("c")
  base = wid * PER_W
  lanes = lax.iota(jnp.int32, 16)

  def start_of(j):
    return jnp.minimum(base + j * CH, P - CH)

  for bin_idx, (feats, coords) in enumerate(((f0, v0), (f1, v1))):
    bin_off = bin_idx * SPATIAL

    def load(j, k, feats=feats, coords=coords):
      s = start_of(j)
      pltpu.async_copy(coords.at[pl.ds(s // 32, CH // 32)],
                       coords_v.at[k], sem_l)
      pltpu.async_copy(feats.at[pl.ds(s, CH)], feat_v.at[k], sem_l)

    def wait_load(j, k, feats=feats, coords=coords):
      s = start_of(j)
      pltpu.make_async_copy(coords.at[pl.ds(s // 32, CH // 32)],
                            coords_v.at[k], sem_l).wait()
      pltpu.make_async_copy(feats.at[pl.ds(s, CH)],
                            feat_v.at[k], sem_l).wait()

    def compute_idx(k, bin_off=bin_off):
      cv = coords_v.at[k]
      for g in range(CH // 16):
        rows4 = (g * 16 + lanes) * 4
        bcol = plsc.load_gather(cv, [rows4 >> 7, rows4 & 127])
        ycol = plsc.load_gather(cv, [rows4 >> 7, (rows4 + 2) & 127])
        xcol = plsc.load_gather(cv, [rows4 >> 7, (rows4 + 3) & 127])
        idx_v[k, pl.ds(g * 16, 16)] = (ycol * NX + xcol) * 2 + bcol + bin_off

    def scatter_and_wait(k):
      pltpu.async_copy(feat_v.at[k], canvas.at[idx_v.at[k]], sem_s).wait()

    load(0, 0)
    load(1, 1)
    for j in range(NCHUNK):
      k = j % 2
      wait_load(j, k)
      compute_idx(k)
      scatter_and_wait(k)
      if j + 2 < NCHUNK:
        load(j + 2, k)


def _sc_scatter(f0, i0, f1, i1, canvas_zeros):
  mesh = plsc.VectorSubcoreMesh(core_axis_name="c", subcore_axis_name="s")
  run = _mpmd._mpmd_map(
      [(mesh, _sc_scatter_body)],
      jax.ShapeDtypeStruct((CANVAS_ROWS, C), jnp.float32),
      input_output_aliases={4: 0},
      compiler_params=pltpu.CompilerParams(
          needs_layout_passes=False, use_tc_tiling_on_sc=False),
      scratch_types=[
          pltpu.VMEM((2, CH // 32, 128), jnp.int32),
          pltpu.VMEM((2, CH), jnp.int32),
          pltpu.VMEM((2, CH, C), jnp.float32),
          pltpu.SemaphoreType.DMA,
          pltpu.SemaphoreType.DMA,
      ],
  )
  return run(f0, i0, f1, i1, canvas_zeros)


S_BLK = 4096
NSB = NY * NX // S_BLK
Y_BLK = S_BLK // NX


def _tc_transpose_body(c0_ref, c1_ref, o0_ref, o1_ref):
  for c_ref, o_ref in ((c0_ref, o0_ref), (c1_ref, o1_ref)):
    x = c_ref[...]
    for b in range(B):
      half = x[:, b * C:(b + 1) * C]
      for yy in range(Y_BLK):
        o_ref[b, :, yy, :] = jnp.transpose(
            half[yy * NX:(yy + 1) * NX, :], (1, 0))


def _tc_transpose(canvas2d):
  in_spec0 = pl.BlockSpec((S_BLK, 2 * C), lambda s: (s, 0))
  in_spec1 = pl.BlockSpec((S_BLK, 2 * C), lambda s: (s + NSB, 0))
  out_spec = pl.BlockSpec((B, C, Y_BLK, NX), lambda s: (0, 0, s, 0))
  return pl.pallas_call(
      _tc_transpose_body,
      grid=(NSB,),
      in_specs=[in_spec0, in_spec1],
      out_specs=[out_spec, out_spec],
      out_shape=[
          jax.ShapeDtypeStruct((B, C, NY, NX), jnp.float32),
          jax.ShapeDtypeStruct((B, C, NY, NX), jnp.float32),
      ],
      compiler_params=pltpu.CompilerParams(
          dimension_semantics=("parallel",),
      ),
  )(canvas2d, canvas2d)


def kernel(pillar_features_bin_0, voxel_coords_bin_0, pillar_features_bin_1,
           voxel_coords_bin_1):
  zeros = jnp.zeros((NBIN * NY * NX, 2 * C), jnp.float32)
  canvas = _sc_scatter(pillar_features_bin_0,
                       voxel_coords_bin_0.reshape(P // 32, 128),
                       pillar_features_bin_1,
                       voxel_coords_bin_1.reshape(P // 32, 128),
                       zeros.reshape(CANVAS_ROWS, C))
  return _tc_transpose(canvas.reshape(NBIN * NY * NX, 2 * C))


# speedup vs baseline: 1.1067x; 1.1067x over previous
import jax
import jax.numpy as jnp
from jax import lax
from jax.experimental import pallas as pl
from jax.experimental.pallas import tpu as pltpu
from jax.experimental.pallas import tpu_sc as plsc
from jax._src.pallas import mpmd as _mpmd

NX = 512
NY = 512
C = 64
P = 100000
B = 2
SPATIAL = B * NY * NX
NBIN = 2
CANVAS_ROWS = NBIN * SPATIAL

NC = 2
NS = 16
NW = NC * NS
PER_W = 3136
CH = 128
NCHUNK = 25


def _sc_scatter_body(f0, v0, f1, v1, zeros_in, canvas,
                     coords_v, idx_v, feat_v, sem_l, sem_s):
  del zeros_in
  wid = lax.axis_index("s") * NC + lax.axis_index("c")
  base = wid * PER_W
  lanes = lax.iota(jnp.int32, 16)

  def start_of(j):
    return jnp.minimum(base + j * CH, P - CH)

  for bin_idx, (feats, coords) in enumerate(((f0, v0), (f1, v1))):
    bin_off = bin_idx * SPATIAL

    def load(j, k, feats=feats, coords=coords):
      s = start_of(j)
      pltpu.async_copy(coords.at[pl.ds(s // 32, CH // 32)],
                       coords_v.at[k], sem_l)
      pltpu.async_copy(feats.at[pl.ds(s, CH)], feat_v.at[k], sem_l)

    def wait_load(j, k, feats=feats, coords=coords):
      s = start_of(j)
      pltpu.make_async_copy(coords.at[pl.ds(s // 32, CH // 32)],
                            coords_v.at[k], sem_l).wait()
      pltpu.make_async_copy(feats.at[pl.ds(s, CH)],
                            feat_v.at[k], sem_l).wait()

    def compute_idx(k, bin_off=bin_off):
      cv = coords_v.at[k]
      for g in range(CH // 16):
        rows4 = (g * 16 + lanes) * 4
        bcol = plsc.load_gather(cv, [rows4 >> 7, rows4 & 127])
        ycol = plsc.load_gather(cv, [rows4 >> 7, (rows4 + 2) & 127])
        xcol = plsc.load_gather(cv, [rows4 >> 7, (rows4 + 3) & 127])
        idx_v[k, pl.ds(g * 16, 16)] = (ycol * NX + xcol) * 2 + bcol + bin_off

    def scatter_and_wait(k):
      pltpu.async_copy(feat_v.at[k], canvas.at[idx_v.at[k]], sem_s).wait()

    load(0, 0)
    load(1, 1)
    for j in range(NCHUNK):
      k = j % 2
      wait_load(j, k)
      compute_idx(k)
      scatter_and_wait(k)
      if j + 2 < NCHUNK:
        load(j + 2, k)


def _sc_scatter(f0, i0, f1, i1, canvas_zeros):
  mesh = plsc.VectorSubcoreMesh(core_axis_name="c", subcore_axis_name="s")
  run = _mpmd._mpmd_map(
      [(mesh, _sc_scatter_body)],
      jax.ShapeDtypeStruct((CANVAS_ROWS, C), jnp.float32),
      input_output_aliases={4: 0},
      compiler_params=pltpu.CompilerParams(
          needs_layout_passes=False, use_tc_tiling_on_sc=False),
      scratch_types=[
          pltpu.VMEM((2, CH // 32, 128), jnp.int32),
          pltpu.VMEM((2, CH), jnp.int32),
          pltpu.VMEM((2, CH, C), jnp.float32),
          pltpu.SemaphoreType.DMA,
          pltpu.SemaphoreType.DMA,
      ],
  )
  return run(f0, i0, f1, i1, canvas_zeros)


S_BLK = 4096
NSB = NY * NX // S_BLK
Y_BLK = S_BLK // NX


def _tc_transpose_body(c0_ref, c1_ref, o0_ref, o1_ref):
  for c_ref, o_ref in ((c0_ref, o0_ref), (c1_ref, o1_ref)):
    xt = jnp.transpose(c_ref[...], (1, 0))
    for b in range(B):
      for yy in range(Y_BLK):
        o_ref[b, :, yy, :] = xt[b * C:(b + 1) * C, yy * NX:(yy + 1) * NX]


def _tc_transpose(canvas2d):
  in_spec0 = pl.BlockSpec((S_BLK, 2 * C), lambda s: (s, 0))
  in_spec1 = pl.BlockSpec((S_BLK, 2 * C), lambda s: (s + NSB, 0))
  out_spec = pl.BlockSpec((B, C, Y_BLK, NX), lambda s: (0, 0, s, 0))
  return pl.pallas_call(
      _tc_transpose_body,
      grid=(NSB,),
      in_specs=[in_spec0, in_spec1],
      out_specs=[out_spec, out_spec],
      out_shape=[
          jax.ShapeDtypeStruct((B, C, NY, NX), jnp.float32),
          jax.ShapeDtypeStruct((B, C, NY, NX), jnp.float32),
      ],
      compiler_params=pltpu.CompilerParams(
          dimension_semantics=("parallel",),
      ),
  )(canvas2d, canvas2d)


def kernel(pillar_features_bin_0, voxel_coords_bin_0, pillar_features_bin_1,
           voxel_coords_bin_1):
  zeros = jnp.zeros((NBIN * NY * NX, 2 * C), jnp.float32)
  canvas = _sc_scatter(pillar_features_bin_0,
                       voxel_coords_bin_0.reshape(P // 32, 128),
                       pillar_features_bin_1,
                       voxel_coords_bin_1.reshape(P // 32, 128),
                       zeros.reshape(CANVAS_ROWS, C))
  return _tc_transpose(canvas.reshape(NBIN * NY * NX, 2 * C))


# speedup vs baseline: 1.1211x; 1.0130x over previous
import functools

import jax
import jax.numpy as jnp
from jax import lax
from jax.experimental import pallas as pl
from jax.experimental.pallas import tpu as pltpu
from jax.experimental.pallas import tpu_sc as plsc
from jax._src.pallas import mpmd as _mpmd

NX = 512
NY = 512
C = 64
P = 100000
B = 2
SPATIAL = B * NY * NX
NBIN = 2
CANVAS_ROWS = NBIN * SPATIAL

NC = 2
NS = 16
NW = NC * NS
PER_W = 3136
CH = 128
NCHUNK = 25


def _sc_scatter_body(bin_idx, feats, coords, zeros_in, canvas,
                     coords_v, idx_v, feat_v, sem_l, sem_s):
  del zeros_in
  wid = lax.axis_index("s") * NC + lax.axis_index("c")
  base = wid * PER_W
  lanes = lax.iota(jnp.int32, 16)

  def start_of(j):
    return jnp.minimum(base + j * CH, P - CH)

  if True:
    bin_off = bin_idx * SPATIAL

    def load(j, k, feats=feats, coords=coords):
      s = start_of(j)
      pltpu.async_copy(coords.at[pl.ds(s // 32, CH // 32)],
                       coords_v.at[k], sem_l)
      pltpu.async_copy(feats.at[pl.ds(s, CH)], feat_v.at[k], sem_l)

    def wait_load(j, k, feats=feats, coords=coords):
      s = start_of(j)
      pltpu.make_async_copy(coords.at[pl.ds(s // 32, CH // 32)],
                            coords_v.at[k], sem_l).wait()
      pltpu.make_async_copy(feats.at[pl.ds(s, CH)],
                            feat_v.at[k], sem_l).wait()

    def compute_idx(k, bin_off=bin_off):
      cv = coords_v.at[k]
      for g in range(CH // 16):
        rows4 = (g * 16 + lanes) * 4
        bcol = plsc.load_gather(cv, [rows4 >> 7, rows4 & 127])
        ycol = plsc.load_gather(cv, [rows4 >> 7, (rows4 + 2) & 127])
        xcol = plsc.load_gather(cv, [rows4 >> 7, (rows4 + 3) & 127])
        idx_v[k, pl.ds(g * 16, 16)] = (ycol * NX + xcol) * 2 + bcol + bin_off

    def scatter_and_wait(k):
      pltpu.async_copy(feat_v.at[k], canvas.at[idx_v.at[k]], sem_s).wait()

    load(0, 0)
    load(1, 1)
    for j in range(NCHUNK):
      k = j % 2
      wait_load(j, k)
      compute_idx(k)
      scatter_and_wait(k)
      if j + 2 < NCHUNK:
        load(j + 2, k)


def _sc_scatter(bin_idx, feats, coords, canvas_in):
  mesh = plsc.VectorSubcoreMesh(core_axis_name="c", subcore_axis_name="s")
  run = _mpmd._mpmd_map(
      [(mesh, functools.partial(_sc_scatter_body, bin_idx))],
      jax.ShapeDtypeStruct((CANVAS_ROWS, C), jnp.float32),
      input_output_aliases={2: 0},
      compiler_params=pltpu.CompilerParams(
          needs_layout_passes=False, use_tc_tiling_on_sc=False),
      scratch_types=[
          pltpu.VMEM((2, CH // 32, 128), jnp.int32),
          pltpu.VMEM((2, CH), jnp.int32),
          pltpu.VMEM((2, CH, C), jnp.float32),
          pltpu.SemaphoreType.DMA,
          pltpu.SemaphoreType.DMA,
      ],
  )
  return run(feats, coords, canvas_in)


S_BLK = 4096
NSB = NY * NX // S_BLK
Y_BLK = S_BLK // NX


def _tc_transpose_body(c0_ref, c1_ref, o0_ref, o1_ref):
  for c_ref, o_ref in ((c0_ref, o0_ref), (c1_ref, o1_ref)):
    xt = jnp.transpose(c_ref[...], (1, 0))
    for b in range(B):
      for yy in range(Y_BLK):
        o_ref[b, :, yy, :] = xt[b * C:(b + 1) * C, yy * NX:(yy + 1) * NX]


def _tc_transpose(canvas2d):
  in_spec0 = pl.BlockSpec((S_BLK, 2 * C), lambda s: (s, 0))
  in_spec1 = pl.BlockSpec((S_BLK, 2 * C), lambda s: (s + NSB, 0))
  out_spec = pl.BlockSpec((B, C, Y_BLK, NX), lambda s: (0, 0, s, 0))
  return pl.pallas_call(
      _tc_transpose_body,
      grid=(NSB,),
      in_specs=[in_spec0, in_spec1],
      out_specs=[out_spec, out_spec],
      out_shape=[
          jax.ShapeDtypeStruct((B, C, NY, NX), jnp.float32),
          jax.ShapeDtypeStruct((B, C, NY, NX), jnp.float32),
      ],
      compiler_params=pltpu.CompilerParams(
          dimension_semantics=("parallel",),
      ),
  )(canvas2d, canvas2d)


def kernel(pillar_features_bin_0, voxel_coords_bin_0, pillar_features_bin_1,
           voxel_coords_bin_1):
  zeros = jnp.zeros((NBIN * NY * NX, 2 * C), jnp.float32)
  canvas = _sc_scatter(0, pillar_features_bin_0,
                       voxel_coords_bin_0.reshape(P // 32, 128),
                       zeros.reshape(CANVAS_ROWS, C))
  canvas = _sc_scatter(1, pillar_features_bin_1,
                       voxel_coords_bin_1.reshape(P // 32, 128),
                       canvas)
  return _tc_transpose(canvas.reshape(NBIN * NY * NX, 2 * C))


# speedup vs baseline: 1.1956x; 1.0664x over previous
import functools

import jax
import jax.numpy as jnp
from jax import lax
from jax.experimental import pallas as pl
from jax.experimental.pallas import tpu as pltpu
from jax.experimental.pallas import tpu_sc as plsc
from jax._src.pallas import mpmd as _mpmd

NX = 512
NY = 512
C = 64
P = 100000
B = 2
SPATIAL = B * NY * NX
NBIN = 2
CANVAS_ROWS = NBIN * SPATIAL

NC = 2
NS = 16
NW = NC * NS
PER_W = 3136
CH = 128
NCHUNK = 25


MEMSET_ROWS = 1024
MEMSET_PER_TILE = CANVAS_ROWS // NW


def _sc_memset_body(canvas, zbuf, sem):
  wid = lax.axis_index("s") * NC + lax.axis_index("c")
  base = wid * MEMSET_PER_TILE
  zero = jnp.zeros((16,), jnp.float32)

  def zrow(i, _):
    for l in range(C // 16):
      zbuf[i, pl.ds(l * 16, 16)] = zero
    return ()

  lax.fori_loop(0, MEMSET_ROWS, zrow, ())
  n = MEMSET_PER_TILE // MEMSET_ROWS
  for i in range(n):
    pltpu.async_copy(zbuf, canvas.at[pl.ds(base + i * MEMSET_ROWS,
                                           MEMSET_ROWS)], sem)
  for i in range(n):
    pltpu.make_async_copy(zbuf, canvas.at[pl.ds(base + i * MEMSET_ROWS,
                                                MEMSET_ROWS)], sem).wait()


def _sc_memset():
  mesh = plsc.VectorSubcoreMesh(core_axis_name="c", subcore_axis_name="s")
  run = _mpmd._mpmd_map(
      [(mesh, _sc_memset_body)],
      jax.ShapeDtypeStruct((CANVAS_ROWS, C), jnp.float32),
      compiler_params=pltpu.CompilerParams(
          needs_layout_passes=False, use_tc_tiling_on_sc=False),
      scratch_types=[
          pltpu.VMEM((MEMSET_ROWS, C), jnp.float32),
          pltpu.SemaphoreType.DMA,
      ],
  )
  return run()


def _sc_scatter_body(bin_idx, feats, coords, zeros_in, canvas,
                     coords_v, idx_v, feat_v, sem_l, sem_s):
  del zeros_in
  wid = lax.axis_index("s") * NC + lax.axis_index("c")
  base = wid * PER_W
  lanes = lax.iota(jnp.int32, 16)

  def start_of(j):
    return jnp.minimum(base + j * CH, P - CH)

  if True:
    bin_off = bin_idx * SPATIAL

    def load(j, k, feats=feats, coords=coords):
      s = start_of(j)
      pltpu.async_copy(coords.at[pl.ds(s // 32, CH // 32)],
                       coords_v.at[k], sem_l)
      pltpu.async_copy(feats.at[pl.ds(s, CH)], feat_v.at[k], sem_l)

    def wait_load(j, k, feats=feats, coords=coords):
      s = start_of(j)
      pltpu.make_async_copy(coords.at[pl.ds(s // 32, CH // 32)],
                            coords_v.at[k], sem_l).wait()
      pltpu.make_async_copy(feats.at[pl.ds(s, CH)],
                            feat_v.at[k], sem_l).wait()

    def compute_idx(k, bin_off=bin_off):
      cv = coords_v.at[k]
      for g in range(CH // 16):
        rows4 = (g * 16 + lanes) * 4
        bcol = plsc.load_gather(cv, [rows4 >> 7, rows4 & 127])
        ycol = plsc.load_gather(cv, [rows4 >> 7, (rows4 + 2) & 127])
        xcol = plsc.load_gather(cv, [rows4 >> 7, (rows4 + 3) & 127])
        idx_v[k, pl.ds(g * 16, 16)] = (ycol * NX + xcol) * 2 + bcol + bin_off

    def scatter_and_wait(k):
      pltpu.async_copy(feat_v.at[k], canvas.at[idx_v.at[k]], sem_s).wait()

    load(0, 0)
    load(1, 1)
    for j in range(NCHUNK):
      k = j % 2
      wait_load(j, k)
      compute_idx(k)
      scatter_and_wait(k)
      if j + 2 < NCHUNK:
        load(j + 2, k)


def _sc_scatter(bin_idx, feats, coords, canvas_in):
  mesh = plsc.VectorSubcoreMesh(core_axis_name="c", subcore_axis_name="s")
  run = _mpmd._mpmd_map(
      [(mesh, functools.partial(_sc_scatter_body, bin_idx))],
      jax.ShapeDtypeStruct((CANVAS_ROWS, C), jnp.float32),
      input_output_aliases={2: 0},
      compiler_params=pltpu.CompilerParams(
          needs_layout_passes=False, use_tc_tiling_on_sc=False),
      scratch_types=[
          pltpu.VMEM((2, CH // 32, 128), jnp.int32),
          pltpu.VMEM((2, CH), jnp.int32),
          pltpu.VMEM((2, CH, C), jnp.float32),
          pltpu.SemaphoreType.DMA,
          pltpu.SemaphoreType.DMA,
      ],
  )
  return run(feats, coords, canvas_in)


S_BLK = 4096
NSB = NY * NX // S_BLK
Y_BLK = S_BLK // NX


def _tc_transpose_body(c0_ref, c1_ref, o0_ref, o1_ref):
  for c_ref, o_ref in ((c0_ref, o0_ref), (c1_ref, o1_ref)):
    xt = jnp.transpose(c_ref[...], (1, 0))
    for b in range(B):
      for yy in range(Y_BLK):
        o_ref[b, :, yy, :] = xt[b * C:(b + 1) * C, yy * NX:(yy + 1) * NX]


def _tc_transpose(canvas2d):
  in_spec0 = pl.BlockSpec((S_BLK, 2 * C), lambda s: (s, 0))
  in_spec1 = pl.BlockSpec((S_BLK, 2 * C), lambda s: (s + NSB, 0))
  out_spec = pl.BlockSpec((B, C, Y_BLK, NX), lambda s: (0, 0, s, 0))
  return pl.pallas_call(
      _tc_transpose_body,
      grid=(NSB,),
      in_specs=[in_spec0, in_spec1],
      out_specs=[out_spec, out_spec],
      out_shape=[
          jax.ShapeDtypeStruct((B, C, NY, NX), jnp.float32),
          jax.ShapeDtypeStruct((B, C, NY, NX), jnp.float32),
      ],
      compiler_params=pltpu.CompilerParams(
          dimension_semantics=("parallel",),
      ),
  )(canvas2d, canvas2d)


def kernel(pillar_features_bin_0, voxel_coords_bin_0, pillar_features_bin_1,
           voxel_coords_bin_1):
  canvas = _sc_scatter(0, pillar_features_bin_0,
                       voxel_coords_bin_0.reshape(P // 32, 128),
                       _sc_memset())
  canvas = _sc_scatter(1, pillar_features_bin_1,
                       voxel_coords_bin_1.reshape(P // 32, 128),
                       canvas)
  return _tc_transpose(canvas.reshape(NBIN * NY * NX, 2 * C))


# speedup vs baseline: 1.2061x; 1.0089x over previous
import functools

import jax
import jax.numpy as jnp
from jax import lax
from jax.experimental import pallas as pl
from jax.experimental.pallas import tpu as pltpu
from jax.experimental.pallas import tpu_sc as plsc
from jax._src.pallas import mpmd as _mpmd

NX = 512
NY = 512
C = 64
P = 100000
B = 2
SPATIAL = B * NY * NX
NBIN = 2
CANVAS_ROWS = NBIN * SPATIAL

NC = 2
NS = 16
NW = NC * NS
PER_W = 3136
CH = 128
NCHUNK = 25


MEMSET_ROWS = 1024
MEMSET_PER_TILE = CANVAS_ROWS // NW


def _sc_memset_body(canvas, zbuf, sem):
  wid = lax.axis_index("s") * NC + lax.axis_index("c")
  base = wid * MEMSET_PER_TILE
  zero = jnp.zeros((16,), jnp.float32)

  def zrow(i, _):
    for l in range(C // 16):
      zbuf[i, pl.ds(l * 16, 16)] = zero
    return ()

  lax.fori_loop(0, MEMSET_ROWS, zrow, ())
  n = MEMSET_PER_TILE // MEMSET_ROWS
  for i in range(n):
    pltpu.async_copy(zbuf, canvas.at[pl.ds(base + i * MEMSET_ROWS,
                                           MEMSET_ROWS)], sem)
  for i in range(n):
    pltpu.make_async_copy(zbuf, canvas.at[pl.ds(base + i * MEMSET_ROWS,
                                                MEMSET_ROWS)], sem).wait()


def _sc_memset():
  mesh = plsc.VectorSubcoreMesh(core_axis_name="c", subcore_axis_name="s")
  run = _mpmd._mpmd_map(
      [(mesh, _sc_memset_body)],
      jax.ShapeDtypeStruct((CANVAS_ROWS, C), jnp.float32),
      compiler_params=pltpu.CompilerParams(
          needs_layout_passes=False, use_tc_tiling_on_sc=False),
      scratch_types=[
          pltpu.VMEM((MEMSET_ROWS, C), jnp.float32),
          pltpu.SemaphoreType.DMA,
      ],
  )
  return run()


def _sc_scatter_body(bin_idx, feats, coords, zeros_in, canvas,
                     coords_v, idx_v, feat_v, sem_l, sem_s):
  del zeros_in
  wid = lax.axis_index("s") * NC + lax.axis_index("c")
  base = wid * PER_W
  lanes = lax.iota(jnp.int32, 16)

  def start_of(j):
    return jnp.minimum(base + j * CH, P - CH)

  if True:
    bin_off = bin_idx * SPATIAL

    def load(j, k, feats=feats, coords=coords):
      s = start_of(j)
      pltpu.async_copy(coords.at[pl.ds(s // 32, CH // 32)],
                       coords_v.at[k], sem_l)
      pltpu.async_copy(feats.at[pl.ds(s, CH)], feat_v.at[k], sem_l)

    def wait_load(j, k, feats=feats, coords=coords):
      s = start_of(j)
      pltpu.make_async_copy(coords.at[pl.ds(s // 32, CH // 32)],
                            coords_v.at[k], sem_l).wait()
      pltpu.make_async_copy(feats.at[pl.ds(s, CH)],
                            feat_v.at[k], sem_l).wait()

    def compute_idx(k, bin_off=bin_off):
      cv = coords_v.at[k]
      for g in range(CH // 16):
        rows4 = (g * 16 + lanes) * 4
        bcol = plsc.load_gather(cv, [rows4 >> 7, rows4 & 127])
        ycol = plsc.load_gather(cv, [rows4 >> 7, (rows4 + 2) & 127])
        xcol = plsc.load_gather(cv, [rows4 >> 7, (rows4 + 3) & 127])
        idx_v[k, pl.ds(g * 16, 16)] = (ycol * NX + xcol) * 2 + bcol + bin_off

    def scatter_and_wait(k):
      pltpu.async_copy(feat_v.at[k], canvas.at[idx_v.at[k]], sem_s).wait()

    load(0, 0)
    load(1, 1)
    for j in range(NCHUNK):
      k = j % 2
      wait_load(j, k)
      compute_idx(k)
      scatter_and_wait(k)
      if j + 2 < NCHUNK:
        load(j + 2, k)


def _sc_scatter(bin_idx, feats, coords, canvas_in):
  mesh = plsc.VectorSubcoreMesh(core_axis_name="c", subcore_axis_name="s")
  run = _mpmd._mpmd_map(
      [(mesh, functools.partial(_sc_scatter_body, bin_idx))],
      jax.ShapeDtypeStruct((CANVAS_ROWS, C), jnp.float32),
      input_output_aliases={2: 0},
      compiler_params=pltpu.CompilerParams(
          needs_layout_passes=False, use_tc_tiling_on_sc=False),
      scratch_types=[
          pltpu.VMEM((2, CH // 32, 128), jnp.int32),
          pltpu.VMEM((2, CH), jnp.int32),
          pltpu.VMEM((2, CH, C), jnp.float32),
          pltpu.SemaphoreType.DMA,
          pltpu.SemaphoreType.DMA,
      ],
  )
  return run(feats, coords, canvas_in)


S_BLK = 8192
NSB = NY * NX // S_BLK
Y_BLK = S_BLK // NX


def _tc_transpose_body(c0_ref, c1_ref, o0_ref, o1_ref):
  for c_ref, o_ref in ((c0_ref, o0_ref), (c1_ref, o1_ref)):
    xt = jnp.transpose(c_ref[...], (1, 0))
    for b in range(B):
      for yy in range(Y_BLK):
        o_ref[b, :, yy, :] = xt[b * C:(b + 1) * C, yy * NX:(yy + 1) * NX]


def _tc_transpose(canvas2d):
  in_spec0 = pl.BlockSpec((S_BLK, 2 * C), lambda s: (s, 0))
  in_spec1 = pl.BlockSpec((S_BLK, 2 * C), lambda s: (s + NSB, 0))
  out_spec = pl.BlockSpec((B, C, Y_BLK, NX), lambda s: (0, 0, s, 0))
  return pl.pallas_call(
      _tc_transpose_body,
      grid=(NSB,),
      in_specs=[in_spec0, in_spec1],
      out_specs=[out_spec, out_spec],
      out_shape=[
          jax.ShapeDtypeStruct((B, C, NY, NX), jnp.float32),
          jax.ShapeDtypeStruct((B, C, NY, NX), jnp.float32),
      ],
      compiler_params=pltpu.CompilerParams(
          dimension_semantics=("parallel",),
      ),
  )(canvas2d, canvas2d)


def kernel(pillar_features_bin_0, voxel_coords_bin_0, pillar_features_bin_1,
           voxel_coords_bin_1):
  canvas = _sc_scatter(0, pillar_features_bin_0,
                       voxel_coords_bin_0.reshape(P // 32, 128),
                       _sc_memset())
  canvas = _sc_scatter(1, pillar_features_bin_1,
                       voxel_coords_bin_1.reshape(P // 32, 128),
                       canvas)
  return _tc_transpose(canvas.reshape(NBIN * NY * NX, 2 * C))


# speedup vs baseline: 1.2072x; 1.0009x over previous
import functools

import jax
import jax.numpy as jnp
from jax import lax
from jax.experimental import pallas as pl
from jax.experimental.pallas import tpu as pltpu
from jax.experimental.pallas import tpu_sc as plsc
from jax._src.pallas import mpmd as _mpmd

NX = 512
NY = 512
C = 64
P = 100000
B = 2
SPATIAL = B * NY * NX
NBIN = 2
CANVAS_ROWS = NBIN * SPATIAL

NC = 2
NS = 16
NW = NC * NS
PER_W = 3136
CH = 128
NCHUNK = 25


MEMSET_ROWS = 1024
MEMSET_PER_TILE = CANVAS_ROWS // NW


def _sc_memset_body(canvas, zbuf, sem):
  wid = lax.axis_index("s") * NC + lax.axis_index("c")
  base = wid * MEMSET_PER_TILE
  zero = jnp.zeros((16,), jnp.float32)

  def zrow(i, _):
    for l in range(C // 16):
      zbuf[i, pl.ds(l * 16, 16)] = zero
    return ()

  lax.fori_loop(0, MEMSET_ROWS, zrow, ())
  n = MEMSET_PER_TILE // MEMSET_ROWS
  for i in range(n):
    pltpu.async_copy(zbuf, canvas.at[pl.ds(base + i * MEMSET_ROWS,
                                           MEMSET_ROWS)], sem)
  for i in range(n):
    pltpu.make_async_copy(zbuf, canvas.at[pl.ds(base + i * MEMSET_ROWS,
                                                MEMSET_ROWS)], sem).wait()


def _sc_memset():
  mesh = plsc.VectorSubcoreMesh(core_axis_name="c", subcore_axis_name="s")
  run = _mpmd._mpmd_map(
      [(mesh, _sc_memset_body)],
      jax.ShapeDtypeStruct((CANVAS_ROWS, C), jnp.float32),
      compiler_params=pltpu.CompilerParams(
          needs_layout_passes=False, use_tc_tiling_on_sc=False),
      scratch_types=[
          pltpu.VMEM((MEMSET_ROWS, C), jnp.float32),
          pltpu.SemaphoreType.DMA,
      ],
  )
  return run()


def _sc_scatter_body(bin_idx, feats, coords, canvas_in, canvas,
                     coords_v, idx_v, feat_v, sem_l, sem_s):
  del canvas_in
  wid = lax.axis_index("s") * NC + lax.axis_index("c")
  base = wid * PER_W
  lanes = lax.iota(jnp.int32, 16)
  bin_off = bin_idx * SPATIAL

  def start_of(j):
    return jnp.minimum(base + j * CH, P - CH)

  def load(j, k):
    s = start_of(j)
    pltpu.async_copy(coords.at[pl.ds(s // 32, CH // 32)],
                     coords_v.at[k], sem_l)
    pltpu.async_copy(feats.at[pl.ds(s, CH)], feat_v.at[k], sem_l)

  def wait_load(j, k):
    s = start_of(j)
    pltpu.make_async_copy(coords.at[pl.ds(s // 32, CH // 32)],
                          coords_v.at[k], sem_l).wait()
    pltpu.make_async_copy(feats.at[pl.ds(s, CH)],
                          feat_v.at[k], sem_l).wait()

  def compute_idx(k):
    cv = coords_v.at[k]
    for g in range(CH // 16):
      rows4 = (g * 16 + lanes) * 4
      bcol = plsc.load_gather(cv, [rows4 >> 7, rows4 & 127])
      ycol = plsc.load_gather(cv, [rows4 >> 7, (rows4 + 2) & 127])
      xcol = plsc.load_gather(cv, [rows4 >> 7, (rows4 + 3) & 127])
      idx_v[k, pl.ds(g * 16, 16)] = (ycol * NX + xcol) * 2 + bcol + bin_off

  def scatter_and_wait(k):
    pltpu.async_copy(feat_v.at[k], canvas.at[idx_v.at[k]], sem_s).wait()

  load(0, 0)
  load(1, 1)
  for j in range(NCHUNK):
    k = j % 2
    wait_load(j, k)
    compute_idx(k)
    scatter_and_wait(k)
    if j + 2 < NCHUNK:
      load(j + 2, k)


def _sc_scatter(bin_idx, feats, coords, canvas_in):
  mesh = plsc.VectorSubcoreMesh(core_axis_name="c", subcore_axis_name="s")
  run = _mpmd._mpmd_map(
      [(mesh, functools.partial(_sc_scatter_body, bin_idx))],
      jax.ShapeDtypeStruct((CANVAS_ROWS, C), jnp.float32),
      input_output_aliases={2: 0},
      compiler_params=pltpu.CompilerParams(
          needs_layout_passes=False, use_tc_tiling_on_sc=False),
      scratch_types=[
          pltpu.VMEM((2, CH // 32, 128), jnp.int32),
          pltpu.VMEM((2, CH), jnp.int32),
          pltpu.VMEM((2, CH, C), jnp.float32),
          pltpu.SemaphoreType.DMA,
          pltpu.SemaphoreType.DMA,
      ],
  )
  return run(feats, coords, canvas_in)


S_BLK = 8192
NSB = NY * NX // S_BLK
Y_BLK = S_BLK // NX


def _tc_transpose_body(c0_ref, c1_ref, o0_ref, o1_ref):
  for c_ref, o_ref in ((c0_ref, o0_ref), (c1_ref, o1_ref)):
    xt = jnp.transpose(c_ref[...], (1, 0))
    for b in range(B):
      for yy in range(Y_BLK):
        o_ref[b, :, yy, :] = xt[b * C:(b + 1) * C, yy * NX:(yy + 1) * NX]


def _tc_transpose(canvas2d):
  in_spec0 = pl.BlockSpec((S_BLK, 2 * C), lambda s: (s, 0))
  in_spec1 = pl.BlockSpec((S_BLK, 2 * C), lambda s: (s + NSB, 0))
  out_spec = pl.BlockSpec((B, C, Y_BLK, NX), lambda s: (0, 0, s, 0))
  return pl.pallas_call(
      _tc_transpose_body,
      grid=(NSB,),
      in_specs=[in_spec0, in_spec1],
      out_specs=[out_spec, out_spec],
      out_shape=[
          jax.ShapeDtypeStruct((B, C, NY, NX), jnp.float32),
          jax.ShapeDtypeStruct((B, C, NY, NX), jnp.float32),
      ],
      compiler_params=pltpu.CompilerParams(
          dimension_semantics=("parallel",),
      ),
  )(canvas2d, canvas2d)


def kernel(pillar_features_bin_0, voxel_coords_bin_0, pillar_features_bin_1,
           voxel_coords_bin_1):
  canvas = _sc_scatter(0, pillar_features_bin_0,
                       voxel_coords_bin_0.reshape(P // 32, 128),
                       _sc_memset())
  canvas = _sc_scatter(1, pillar_features_bin_1,
                       voxel_coords_bin_1.reshape(P // 32, 128),
                       canvas)
  return _tc_transpose(canvas.reshape(NBIN * NY * NX, 2 * C))
